# Initial kernel scaffold; baseline (speedup 1.0000x reference)
import jax
import jax.numpy as jnp
from jax.experimental import pallas as pl


def kernel(x, y, table):
    raise NotImplementedError("write your pallas kernel here")


# speedup vs baseline: 1.6685x; 1.6685x over previous
import functools

import jax
import jax.numpy as jnp
from jax import lax
from jax.experimental import pallas as pl
from jax.experimental.pallas import tpu as pltpu
from jax.experimental.pallas import tpu_sc as plsc

NC, NS, L = 2, 16, 16
NW = NC * NS
V = 1000
B_TOT = 1024 * 50
BPW = B_TOT // NW
CH = 64
NCHUNK = BPW // CH
GRP = CH // L


def _lse_body(table_ref, lse_ref):
    t = table_ref[...]
    m = jnp.max(t, axis=1, keepdims=True)
    s = jnp.sum(jnp.exp(t - m), axis=1, keepdims=True)
    lse_ref[...] = m + jnp.log(s)


_lse_call = pl.pallas_call(
    _lse_body,
    out_shape=jax.ShapeDtypeStruct((V, 1), jnp.float32),
)

_sc_mesh = plsc.VectorSubcoreMesh(
    core_axis_name="c", subcore_axis_name="s", num_cores=NC, num_subcores=NS
)


@functools.partial(
    pl.kernel,
    out_type=(
        jax.ShapeDtypeStruct((B_TOT, V), jnp.float32),
        jax.ShapeDtypeStruct((NW, L), jnp.float32),
    ),
    mesh=_sc_mesh,
    compiler_params=pltpu.CompilerParams(
        use_tc_tiling_on_sc=False, needs_layout_passes=False
    ),
    scratch_types=[
        pltpu.VMEM((BPW,), jnp.int32),
        pltpu.VMEM((BPW,), jnp.int32),
        pltpu.VMEM((V,), jnp.float32),
        pltpu.VMEM((CH, V), jnp.float32),
        pltpu.VMEM((L,), jnp.float32),
        pltpu.SemaphoreType.DMA,
    ],
)
def _sc_gather(table_hbm, x_hbm, y_hbm, lse_hbm, out_hbm, part_hbm,
               x_v, y_v, lse_v, rows_v, acc_v, sem):
    wid = lax.axis_index("s") * NC + lax.axis_index("c")
    base = wid * BPW
    pltpu.sync_copy(x_hbm.at[pl.ds(base, BPW)], x_v)
    pltpu.sync_copy(y_hbm.at[pl.ds(base, BPW)], y_v)
    pltpu.sync_copy(lse_hbm, lse_v)

    def chunk_body(ci, acc):
        off = pl.multiple_of(ci * CH, 8)
        pltpu.async_copy(
            table_hbm.at[x_v.at[pl.ds(off, CH)]], rows_v, sem
        ).wait()
        pltpu.sync_copy(rows_v, out_hbm.at[pl.ds(base + off, CH)])
        lanes = lax.iota(jnp.int32, L)
        for gi in range(GRP):
            goff = gi * L
            xg = x_v[pl.ds(off + goff, L)]
            cols = y_v[pl.ds(off + goff, L)]
            tgt = plsc.load_gather(rows_v, [lanes + goff, cols])
            lse_g = plsc.load_gather(lse_v, [xg])
            acc = acc + (lse_g - tgt)
        return acc

    acc = lax.fori_loop(0, NCHUNK, chunk_body, jnp.zeros((L,), jnp.float32))
    acc_v[...] = acc
    pltpu.sync_copy(acc_v, part_hbm.at[wid])


def kernel(x, y, table):
    xf = x.reshape(-1).astype(jnp.int32)
    yf = y.reshape(-1).astype(jnp.int32)
    lse = _lse_call(table).reshape(V)
    logits2, parts = _sc_gather(table, xf, yf, lse)
    loss = jnp.sum(parts) / B_TOT
    return (logits2, loss)


# speedup vs baseline: 1.6913x; 1.0137x over previous
import functools

import jax
import jax.numpy as jnp
from jax import lax
from jax.experimental import pallas as pl
from jax.experimental.pallas import tpu as pltpu
from jax.experimental.pallas import tpu_sc as plsc

NC, NS, L = 2, 16, 16
NW = NC * NS
V = 1000
B_TOT = 1024 * 50
BPW = B_TOT // NW
CH = 32
NCHUNK = BPW // CH
NPAIR = NCHUNK // 2
GRP = CH // L


def _lse_body(table_ref, lse_ref):
    t = table_ref[...]
    m = jnp.max(t, axis=1, keepdims=True)
    s = jnp.sum(jnp.exp(t - m), axis=1, keepdims=True)
    lse_ref[...] = m + jnp.log(s)


_lse_call = pl.pallas_call(
    _lse_body,
    out_shape=jax.ShapeDtypeStruct((V, 1), jnp.float32),
)

_sc_mesh = plsc.VectorSubcoreMesh(
    core_axis_name="c", subcore_axis_name="s", num_cores=NC, num_subcores=NS
)


@functools.partial(
    pl.kernel,
    out_type=(
        jax.ShapeDtypeStruct((B_TOT, V), jnp.float32),
        jax.ShapeDtypeStruct((NW, L), jnp.float32),
    ),
    mesh=_sc_mesh,
    compiler_params=pltpu.CompilerParams(
        use_tc_tiling_on_sc=False, needs_layout_passes=False
    ),
    scratch_types=[
        pltpu.VMEM((BPW,), jnp.int32),
        pltpu.VMEM((BPW,), jnp.int32),
        pltpu.VMEM((V,), jnp.float32),
        pltpu.VMEM((CH, V), jnp.float32),
        pltpu.VMEM((CH, V), jnp.float32),
        pltpu.VMEM((L,), jnp.float32),
        pltpu.SemaphoreType.DMA,
        pltpu.SemaphoreType.DMA,
        pltpu.SemaphoreType.DMA,
    ],
)
def _sc_gather(table_hbm, x_hbm, y_hbm, lse_hbm, out_hbm, part_hbm,
               x_v, y_v, lse_v, rows0, rows1, acc_v,
               sem_in, sem_out0, sem_out1):
    wid = lax.axis_index("s") * NC + lax.axis_index("c")
    base = wid * BPW
    pltpu.sync_copy(x_hbm.at[pl.ds(base, BPW)], x_v)
    pltpu.sync_copy(y_hbm.at[pl.ds(base, BPW)], y_v)
    pltpu.sync_copy(lse_hbm, lse_v)
    bufs = ((rows0, sem_out0), (rows1, sem_out1))
    lanes = lax.iota(jnp.int32, L)

    def pair_body(g, acc):
        for b in range(2):
            ci = 2 * g + b
            off = pl.multiple_of(ci * CH, 8)
            rows_v, sem_out = bufs[b]

            @pl.when(g >= 1)
            def _():
                pltpu.make_async_copy(
                    rows_v, out_hbm.at[pl.ds(base, CH)], sem_out
                ).wait()

            pltpu.async_copy(
                table_hbm.at[x_v.at[pl.ds(off, CH)]], rows_v, sem_in
            ).wait()
            pltpu.async_copy(rows_v, out_hbm.at[pl.ds(base + off, CH)], sem_out)
            for gi in range(GRP):
                goff = gi * L
                xg = x_v[pl.ds(off + goff, L)]
                cols = y_v[pl.ds(off + goff, L)]
                tgt = plsc.load_gather(rows_v, [lanes + goff, cols])
                lse_g = plsc.load_gather(lse_v, [xg])
                acc = acc + (lse_g - tgt)
        return acc

    acc = lax.fori_loop(0, NPAIR, pair_body, jnp.zeros((L,), jnp.float32))
    for rows_v, sem_out in bufs:
        pltpu.make_async_copy(
            rows_v, out_hbm.at[pl.ds(base, CH)], sem_out
        ).wait()
    acc_v[...] = acc
    pltpu.sync_copy(acc_v, part_hbm.at[wid])


def kernel(x, y, table):
    xf = x.reshape(-1).astype(jnp.int32)
    yf = y.reshape(-1).astype(jnp.int32)
    lse = _lse_call(table).reshape(V)
    logits2, parts = _sc_gather(table, xf, yf, lse)
    loss = jnp.sum(parts) / B_TOT
    return (logits2, loss)
